# baseline (speedup 1.0000x reference)
import functools

import jax
import jax.numpy as jnp
from jax import lax
from jax.experimental import pallas as pl
from jax.experimental.pallas import tpu as pltpu
from jax.experimental.pallas import tpu_sc as plsc

N_USERS = 25000
M_ITEMS = 25000
N_NODES = N_USERS + M_ITEMS
N_EDGES = 800000
D = 64
HD = D // 2
NC = 2
NS = 16
CH = 128
NCHG = N_EDGES // CH
NROUND = (NCHG + NS - 1) // NS
NBUF = 6
NIT = 64
RCH = 40
NRC = N_NODES // RCH
URC = N_USERS // RCH
KMAX = (NRC + NS - 1) // NS
KIT = 19

_mesh = plsc.VectorSubcoreMesh(core_axis_name="c", subcore_axis_name="s")


@functools.partial(
    pl.kernel,
    mesh=_mesh,
    compiler_params=pltpu.CompilerParams(use_tc_tiling_on_sc=False),
    out_type=[
        jax.ShapeDtypeStruct((NC, N_USERS, HD), jnp.float32),
        jax.ShapeDtypeStruct((NC, M_ITEMS, HD), jnp.float32),
        jax.ShapeDtypeStruct((NC * N_NODES, HD), jnp.float32),
        jax.ShapeDtypeStruct((NC * N_NODES, HD), jnp.float32),
    ],
    scratch_types=[
        pltpu.VMEM_SHARED((N_NODES, HD), jnp.float32),
        pltpu.VMEM((3, CH), jnp.int32),
        pltpu.VMEM((3, CH), jnp.int32),
        pltpu.VMEM((3, CH), jnp.int32),
        pltpu.VMEM((3, CH), jnp.int32),
        pltpu.VMEM((3, CH), jnp.int32),
        pltpu.VMEM((3, CH), jnp.int32),
        pltpu.VMEM((CH, HD), jnp.float32),
        pltpu.VMEM((CH, HD), jnp.float32),
        pltpu.VMEM((CH, HD), jnp.float32),
        pltpu.VMEM((CH, HD), jnp.float32),
        pltpu.VMEM((CH, HD), jnp.float32),
        pltpu.VMEM((CH, HD), jnp.float32),
        pltpu.VMEM((RCH, HD), jnp.float32),
        pltpu.VMEM((RCH, HD), jnp.float32),
        pltpu.VMEM((RCH, HD), jnp.float32),
        pltpu.SemaphoreType.DMA,
        pltpu.SemaphoreType.DMA,
        pltpu.SemaphoreType.DMA,
        pltpu.SemaphoreType.DMA,
        pltpu.SemaphoreType.DMA,
        pltpu.SemaphoreType.DMA,
        pltpu.SemaphoreType.DMA,
        pltpu.SemaphoreType.DMA,
        pltpu.SemaphoreType.DMA,
        pltpu.SemaphoreType.DMA,
        pltpu.SemaphoreType.DMA,
        pltpu.SemaphoreType.DMA,
        pltpu.SemaphoreType.DMA,
        pltpu.SemaphoreType.DMA,
        pltpu.SemaphoreType.DMA,
        pltpu.SemaphoreType.DMA,
        pltpu.SemaphoreType.DMA,
        pltpu.SemaphoreType.DMA,
    ],
)
def _gcn(tab0, epk, uo, io, t1, t2,
         acc, eb0, eb1, eb2, eb3, eb4, eb5,
         rv0, rv1, rv2, rv3, rv4, rv5, zbuf, b0, b1,
         se0, se1, se2, se3, se4, se5,
         sg0, sg1, sg2, sg3, sg4, sg5,
         ss0, ss1, ss2, ss3, ss4, ss5):
    c = lax.axis_index("c")
    s = lax.axis_index("s")
    cbase = c * N_NODES
    EB = [eb0, eb1, eb2, eb3, eb4, eb5]
    RV = [rv0, rv1, rv2, rv3, rv4, rv5]
    SE = [se0, se1, se2, se3, se4, se5]
    SG = [sg0, sg1, sg2, sg3, sg4, sg5]
    SS = [ss0, ss1, ss2, ss3, ss4, ss5]
    BO = [b0, b1]

    def zb(i, carry):
        zbuf[i, pl.ds(0, 16)] = jnp.zeros((16,), jnp.float32)
        zbuf[i, pl.ds(16, 16)] = jnp.zeros((16,), jnp.float32)
        return carry

    lax.fori_loop(0, RCH, zb, 0)

    def zero_acc():
        def rr(k, carry):
            cid = s + k * NS

            @pl.when(cid < NRC)
            def _():
                r = pl.multiple_of(cid * RCH, 8)
                pltpu.sync_copy(zbuf, acc.at[pl.ds(r, RCH), :])

            return carry

        lax.fori_loop(0, KMAX, rr, 0)

    def layer(tab):
        def valid(k):
            return s + k * NS < NCHG

        def edge_desc(k, p):
            cid = s + k * NS
            return pltpu.make_async_copy(epk.at[c, cid], EB[p], SE[p])

        def gather_start(p):
            pltpu.make_async_copy(tab.at[EB[p].at[0]], RV[p], SG[p]).start()

        def gather_wait(p):
            pltpu.make_async_copy(tab.at[EB[p].at[0]], RV[p], SG[p]).wait()

        def scatter_start(p):
            pltpu.make_async_copy(
                RV[p], acc.at[EB[p].at[1]], SS[p]).start(add=True)

        def scatter_wait(p):
            pltpu.make_async_copy(RV[p], acc.at[EB[p].at[1]], SS[p]).wait()

        def scale(p):
            def body(i, carry2):
                base = i * 16
                wvec = lax.bitcast_convert_type(
                    EB[p][2, pl.ds(base, 16)], jnp.float32)
                for j in range(16):
                    wi = wvec[j]
                    e = base + j
                    RV[p][e, pl.ds(0, 16)] = RV[p][e, pl.ds(0, 16)] * wi
                    RV[p][e, pl.ds(16, 16)] = RV[p][e, pl.ds(16, 16)] * wi
                return carry2

            lax.fori_loop(0, CH // 16, body, 0)

        for j in range(4):
            edge_desc(j, j).start()
        edge_desc(0, 0).wait()
        gather_start(0)
        edge_desc(1, 1).wait()
        gather_start(1)

        def it(i, carry):
            k0 = i * NBUF
            for b in range(NBUF):
                k = k0 + b

                @pl.when(k >= 2)
                def _(b=b):
                    scatter_wait((b + 4) % NBUF)

                edge_desc(k + 4, (b + 4) % NBUF).start()
                edge_desc(k + 2, (b + 2) % NBUF).wait()
                gather_start((b + 2) % NBUF)
                gather_wait(b)
                scale(b)
                scatter_start(b)
            return carry

        lax.fori_loop(0, NIT, it, 0)

        def tail(k, do_e, do_g):
            b = k % NBUF
            scatter_wait((b + 4) % NBUF)
            if do_e == "always":
                edge_desc(k + 4, (b + 4) % NBUF).start()
            elif do_e == "guard":
                @pl.when(valid(k + 4))
                def _(k=k, b=b):
                    edge_desc(k + 4, (b + 4) % NBUF).start()
            if do_g == "always":
                edge_desc(k + 2, (b + 2) % NBUF).wait()
                gather_start((b + 2) % NBUF)
            elif do_g == "guard":
                @pl.when(valid(k + 2))
                def _(k=k, b=b):
                    edge_desc(k + 2, (b + 2) % NBUF).wait()
                    gather_start((b + 2) % NBUF)
            gather_wait(b)
            scale(b)
            scatter_start(b)

        tail(384, "always", "always")
        tail(385, "always", "always")
        tail(386, "guard", "always")
        tail(387, "none", "always")
        tail(388, "none", "guard")
        tail(389, "none", "none")

        scatter_wait(4)

        @pl.when(valid(390))
        def _():
            gather_wait(0)
            scale(0)
            scatter_start(0)

        scatter_wait(5)

        @pl.when(valid(390))
        def _():
            scatter_wait(0)

    def copy_out_and_zero(thbm):
        def valid(k):
            return s + k * NS < NRC

        def rowoff(k):
            return pl.multiple_of(cbase + (s + k * NS) * RCH, 8)

        def accoff(k):
            return pl.multiple_of((s + k * NS) * RCH, 8)

        def load_desc(k, p):
            return pltpu.make_async_copy(
                acc.at[pl.ds(accoff(k), RCH), :],
                RV[p].at[pl.ds(0, RCH), :], SG[p])

        def store_desc(k, p):
            return pltpu.make_async_copy(
                RV[p].at[pl.ds(0, RCH), :],
                thbm.at[pl.ds(rowoff(k), RCH), :], SS[p])

        def zero_desc(k, p):
            return pltpu.make_async_copy(
                zbuf, acc.at[pl.ds(accoff(k), RCH), :], SE[p])

        load_desc(0, 0).start()
        load_desc(1, 1).start()

        def step(k, b, guard_lo):
            load_desc(k, b).wait()
            store_desc(k, b).start()
            zero_desc(k, b).start()
            q = (b + 2) % 4

            def free_and_load(k=k, q=q):
                store_desc(k - 2, q).wait()
                zero_desc(k - 2, q).wait()
                load_desc(k + 2, q).start()

            if guard_lo:
                @pl.when(k >= 2)
                def _():
                    free_and_load()

                @pl.when(k < 2)
                def _(k=k, q=q):
                    load_desc(k + 2, q).start()
            else:
                free_and_load()

        def it(i, carry):
            k0 = i * 4
            for b in range(4):
                step(k0 + b, b, guard_lo=True)
            return carry

        lax.fori_loop(0, KIT, it, 0)

        load_desc(76, 0).wait()
        store_desc(76, 0).start()
        zero_desc(76, 0).start()
        store_desc(74, 2).wait()
        zero_desc(74, 2).wait()

        @pl.when(valid(78))
        def _():
            load_desc(78, 2).start()

        load_desc(77, 1).wait()
        store_desc(77, 1).start()
        zero_desc(77, 1).start()
        store_desc(75, 3).wait()
        zero_desc(75, 3).wait()

        @pl.when(valid(78))
        def _():
            load_desc(78, 2).wait()
            store_desc(78, 2).start()
            zero_desc(78, 2).start()
            store_desc(78, 2).wait()
            zero_desc(78, 2).wait()

        store_desc(76, 0).wait()
        zero_desc(76, 0).wait()
        store_desc(77, 1).wait()
        zero_desc(77, 1).wait()

    def final(tab):
        def valid(k):
            return s + k * NS < NRC

        def out_desc(k, p):
            cid = s + k * NS
            r = pl.multiple_of(cid * RCH, 8)
            ri = pl.multiple_of(r - N_USERS, 8)
            du = pltpu.make_async_copy(BO[p], uo.at[c, pl.ds(r, RCH), :],
                                       SS[p])
            di = pltpu.make_async_copy(BO[p], io.at[c, pl.ds(ri, RCH), :],
                                       SS[p])
            return du, di

        def wait_out(k, p):
            du, di = out_desc(k, p)
            cid2 = s + k * NS

            @pl.when(cid2 < URC)
            def _():
                du.wait()

            @pl.when(cid2 >= URC)
            def _():
                di.wait()

        def fp(k, p):
            cid = s + k * NS
            r = pl.multiple_of(cid * RCH, 8)
            rt = pl.multiple_of(cbase + r, 8)
            d0 = pltpu.make_async_copy(
                tab.at[pl.ds(rt, RCH), :], RV[0].at[pl.ds(0, RCH), :], SG[0])
            d1 = pltpu.make_async_copy(
                t1.at[pl.ds(rt, RCH), :], RV[1].at[pl.ds(0, RCH), :], SG[1])
            d2 = pltpu.make_async_copy(
                t2.at[pl.ds(rt, RCH), :], RV[2].at[pl.ds(0, RCH), :], SG[2])
            d3 = pltpu.make_async_copy(
                acc.at[pl.ds(r, RCH), :], RV[3].at[pl.ds(0, RCH), :], SG[3])
            for d in (d0, d1, d2, d3):
                d.start()

            @pl.when(k >= 2)
            def _(k=k, p=p):
                wait_out(k - 2, p)

            for d in (d0, d1, d2, d3):
                d.wait()

            def srow(i, carry2):
                for off in (0, 16):
                    v = (RV[0][i, pl.ds(off, 16)] + RV[1][i, pl.ds(off, 16)]
                         + RV[2][i, pl.ds(off, 16)]
                         + RV[3][i, pl.ds(off, 16)])
                    BO[p][i, pl.ds(off, 16)] = v * 0.25
                return carry2

            lax.fori_loop(0, RCH, srow, 0)

            du, di = out_desc(k, p)

            @pl.when(cid < URC)
            def _():
                du.start()

            @pl.when(cid >= URC)
            def _():
                di.start()

        def it(i, carry):
            k0 = i * 2
            for b in range(2):
                k = k0 + b

                @pl.when(valid(k))
                def _(k=k, b=b):
                    fp(k, b)

            return carry

        lax.fori_loop(0, (KMAX + 1) // 2, it, 0)

        for p in range(2):
            last0 = KMAX - 1 if (KMAX - 1) % 2 == p else KMAX - 2

            @pl.when(valid(last0))
            def _(p=p, last0=last0):
                wait_out(last0, p)

            @pl.when(jnp.logical_not(valid(last0)))
            def _(p=p, last0=last0):
                wait_out(last0 - 2, p)

    zero_acc()
    plsc.subcore_barrier()
    layer(tab0)
    plsc.subcore_barrier()
    copy_out_and_zero(t1)
    plsc.subcore_barrier()
    layer(t1)
    plsc.subcore_barrier()
    copy_out_and_zero(t2)
    plsc.subcore_barrier()
    layer(t2)
    plsc.subcore_barrier()
    final(tab0)


def kernel(user_emb, item_emb, edge_index, edge_weight):
    emb = jnp.concatenate([user_emb, item_emb], axis=0)
    tab0 = jnp.concatenate([emb[:, :HD], emb[:, HD:]], axis=0)
    src = edge_index[0].astype(jnp.int32).reshape(NCHG, CH)
    dst = edge_index[1].astype(jnp.int32).reshape(NCHG, CH)
    wbits = lax.bitcast_convert_type(
        edge_weight.astype(jnp.float32), jnp.int32).reshape(NCHG, CH)
    epk0 = jnp.stack([src, dst, wbits], axis=1)
    epk1 = jnp.stack([src + N_NODES, dst, wbits], axis=1)
    epk = jnp.stack([epk0, epk1], axis=0)
    uo, io, *_ = _gcn(tab0, epk)
    user_final = jnp.concatenate([uo[0], uo[1]], axis=1)
    item_final = jnp.concatenate([io[0], io[1]], axis=1)
    return (user_final, item_final)
